# baseline (device time: 8496 ns/iter reference)
import jax
import jax.numpy as jnp
from jax import lax
from jax.experimental import pallas as pl
from jax.experimental.pallas import tpu as pltpu

N_DEV = 4


def kernel(x):
    m, n = x.shape

    def body(x_ref, out_ref, comm_ref, send_sems, recv_sems):
        my = lax.axis_index("i")

        barrier_sem = pltpu.get_barrier_semaphore()
        for d in range(1, N_DEV):
            pl.semaphore_signal(
                barrier_sem,
                inc=1,
                device_id=((my + d) % N_DEV,),
                device_id_type=pl.DeviceIdType.MESH,
            )
        pl.semaphore_wait(barrier_sem, N_DEV - 1)

        x_val = x_ref[:, :]

        tot = x_val
        rows = m
        while rows > 1:
            half = rows // 2
            tot = tot[:half] * tot[half:rows]
            rows = half
        comm_ref[0] = tot
        rdmas = []
        for d in range(1, N_DEV):
            rdma = pltpu.make_async_remote_copy(
                src_ref=comm_ref.at[0],
                dst_ref=comm_ref.at[d],
                send_sem=send_sems.at[d - 1],
                recv_sem=recv_sems.at[d - 1],
                device_id=((my + d) % N_DEV,),
                device_id_type=pl.DeviceIdType.MESH,
            )
            rdma.start()
            rdmas.append(rdma)

        B = 8
        G = m // B
        y3 = x_val.reshape(G, B, n)
        shift = 1
        while shift < B:
            head = jnp.ones((G, shift, n), dtype=y3.dtype)
            y3 = y3 * jnp.concatenate([head, y3[:, : B - shift]], axis=1)
            shift *= 2

        t = y3[:, B - 1]
        shift = 1
        while shift < G:
            head = jnp.ones((shift, n), dtype=t.dtype)
            t = t * jnp.concatenate([head, t[: G - shift]], axis=0)
            shift *= 2

        prefix = jnp.ones((1, n), dtype=x_val.dtype)
        for d in range(1, N_DEV):
            rdmas[d - 1].wait()
            origin = (my - d) % N_DEV
            prefix = prefix * jnp.where(origin < my, comm_ref[d], 1.0)

        eb = jnp.concatenate([prefix, t[: G - 1] * prefix], axis=0)
        out_ref[:, :] = (y3 * eb.reshape(G, 1, n)).reshape(m, n)

    return pl.pallas_call(
        body,
        out_shape=jax.ShapeDtypeStruct((m, n), x.dtype),
        in_specs=[pl.BlockSpec(memory_space=pltpu.VMEM)],
        out_specs=pl.BlockSpec(memory_space=pltpu.VMEM),
        scratch_shapes=[
            pltpu.VMEM((N_DEV, 1, n), x.dtype),
            pltpu.SemaphoreType.DMA((N_DEV - 1,)),
            pltpu.SemaphoreType.DMA((N_DEV - 1,)),
        ],
        compiler_params=pltpu.CompilerParams(collective_id=0),
    )(x)


# device time: 7282 ns/iter; 1.1667x vs baseline; 1.1667x over previous
import jax
import jax.numpy as jnp
from jax import lax
from jax.experimental import pallas as pl
from jax.experimental.pallas import tpu as pltpu

N_DEV = 4


def kernel(x):
    m, n = x.shape

    def body(x_ref, out_ref, comm_ref, send_sems, recv_sems):
        my = lax.axis_index("i")

        barrier_sem = pltpu.get_barrier_semaphore()
        for d in range(1, N_DEV):
            pl.semaphore_signal(
                barrier_sem,
                inc=1,
                device_id=((my + d) % N_DEV,),
                device_id_type=pl.DeviceIdType.MESH,
            )

        x_val = x_ref[:, :]

        tot = x_val
        rows = m
        while rows > 1:
            half = rows // 2
            tot = tot[:half] * tot[half:rows]
            rows = half
        comm_ref[0] = tot

        pl.semaphore_wait(barrier_sem, N_DEV - 1)

        rdmas = []
        for d in range(1, N_DEV):
            rdma = pltpu.make_async_remote_copy(
                src_ref=comm_ref.at[0],
                dst_ref=comm_ref.at[d],
                send_sem=send_sems.at[d - 1],
                recv_sem=recv_sems.at[d - 1],
                device_id=((my + d) % N_DEV,),
                device_id_type=pl.DeviceIdType.MESH,
            )
            rdma.start()
            rdmas.append(rdma)

        y = x_val
        shift = 1
        while shift < m:
            head = jnp.ones((shift, n), dtype=y.dtype)
            y = y * jnp.concatenate([head, y[: m - shift]], axis=0)
            shift *= 2

        prefix = jnp.ones((1, n), dtype=x_val.dtype)
        for d in range(1, N_DEV):
            rdmas[d - 1].wait()
            origin = (my - d) % N_DEV
            prefix = prefix * jnp.where(origin < my, comm_ref[d], 1.0)

        out_ref[:, :] = y * prefix

    return pl.pallas_call(
        body,
        out_shape=jax.ShapeDtypeStruct((m, n), x.dtype),
        in_specs=[pl.BlockSpec(memory_space=pltpu.VMEM)],
        out_specs=pl.BlockSpec(memory_space=pltpu.VMEM),
        scratch_shapes=[
            pltpu.VMEM((N_DEV, 1, n), x.dtype),
            pltpu.SemaphoreType.DMA((N_DEV - 1,)),
            pltpu.SemaphoreType.DMA((N_DEV - 1,)),
        ],
        compiler_params=pltpu.CompilerParams(collective_id=0),
    )(x)
